# baseline (speedup 1.0000x reference)
import jax
import jax.numpy as jnp
from jax import lax
from jax.experimental import pallas as pl
from jax.experimental.pallas import tpu as pltpu
from jax.experimental.pallas import tpu_sc as plsc

B = 8
S = 2048
D = 2048
DK = 256
NODE = 64
L = 64
N = B * NODE

NC = 2
NS = 16
NW = NC * NS
LANES = 16

BAGS_PER_W = N // NW
SBLK = 1024

_MESH = plsc.VectorSubcoreMesh(core_axis_name="c", subcore_axis_name="s",
                               num_cores=NC, num_subcores=NS)


def _energy_body(x_ref, w_ref, b_ref, h_ref, wq_ref, wv_ref, e_ref):
    x = x_ref[0].astype(jnp.bfloat16)
    w = w_ref[...].astype(jnp.bfloat16)
    p = jnp.dot(x, w, preferred_element_type=jnp.float32)
    t = jnp.dot(h_ref[0].astype(jnp.bfloat16),
                wq_ref[...].astype(jnp.bfloat16),
                preferred_element_type=jnp.float32)
    tmp = jnp.tanh(p + b_ref[...] + t)
    e_ref[0] = jnp.dot(tmp.astype(jnp.bfloat16),
                       wv_ref[...].astype(jnp.bfloat16),
                       preferred_element_type=jnp.float32)


def _energy(seq_output, W_pre, b_pre2, hidden3, W_q, wv2):
    return pl.pallas_call(
        _energy_body,
        grid=(B, S // SBLK),
        in_specs=[
            pl.BlockSpec((1, SBLK, D), lambda b, s: (b, s, 0)),
            pl.BlockSpec((D, DK), lambda b, s: (0, 0)),
            pl.BlockSpec((1, DK), lambda b, s: (0, 0)),
            pl.BlockSpec((1, 1, D), lambda b, s: (b, 0, 0)),
            pl.BlockSpec((D, DK), lambda b, s: (0, 0)),
            pl.BlockSpec((DK, 1), lambda b, s: (0, 0)),
        ],
        out_specs=pl.BlockSpec((1, SBLK, 1), lambda b, s: (b, s, 0)),
        out_shape=jax.ShapeDtypeStruct((B, S, 1), jnp.float32),
        compiler_params=pltpu.CompilerParams(
            vmem_limit_bytes=100 * 1024 * 1024),
    )(seq_output, W_pre, b_pre2, hidden3, W_q, wv2)


def _sc_attn_body(e_flat, idx_t, a_out, e_row, ix_v, sc_v, acc):
    wid = lax.axis_index("s") * NC + lax.axis_index("c")
    sample = wid // (NW // B)
    pltpu.sync_copy(e_flat.at[pl.ds(sample * S, S)], e_row)
    pltpu.sync_copy(idx_t.at[wid], ix_v)
    zeros = jnp.zeros((LANES,), jnp.float32)

    def zero_row(j, _):
        for i in range(LANES):
            acc[i, pl.ds(j * LANES, LANES)] = zeros
        return 0

    lax.fori_loop(0, S // LANES, zero_row, 0)

    def bag_softmax(i, _):
        base = i * L
        ev = [plsc.load_gather(e_row, [ix_v[pl.ds(base + k * LANES, LANES)]])
              for k in range(L // LANES)]
        m = ev[0]
        for v in ev[1:]:
            m = jnp.maximum(m, v)
        mm = jnp.max(m)
        pv = [jnp.exp(v - mm) for v in ev]
        ssum = pv[0]
        for v in pv[1:]:
            ssum = ssum + v
        rv = (zeros + 1.0) / (zeros + jnp.sum(ssum))
        for k in range(L // LANES):
            sc_v[pl.ds(base + k * LANES, LANES)] = pv[k] * rv
        return 0

    lax.fori_loop(0, BAGS_PER_W, bag_softmax, 0)

    rows16 = lax.iota(jnp.int32, LANES)
    pos0 = rows16 * L

    def scatter_l(l, _):
        pos = pos0 + l
        iv = plsc.load_gather(ix_v, [pos])
        sv = plsc.load_gather(sc_v, [pos])
        plsc.addupdate_scatter(acc, [rows16, iv], sv)
        return 0

    lax.fori_loop(0, L, scatter_l, 0)
    pltpu.sync_copy(acc, a_out.at[pl.ds(wid * BAGS_PER_W, BAGS_PER_W)])


def _sc_attention(e_flat, idx_t):
    k = pl.kernel(
        _sc_attn_body,
        out_type=jax.ShapeDtypeStruct((N, S), jnp.float32),
        mesh=_MESH,
        scratch_types=[
            pltpu.VMEM((S,), jnp.float32),
            pltpu.VMEM((BAGS_PER_W * L,), jnp.int32),
            pltpu.VMEM((BAGS_PER_W * L,), jnp.float32),
            pltpu.VMEM((LANES, S), jnp.float32),
        ],
        compiler_params=pltpu.CompilerParams(use_tc_tiling_on_sc=False,
                                             needs_layout_passes=False,
                                             disable_bounds_checks=True),
    )
    return k(e_flat, idx_t)


def _ctx_body(a_ref, x_ref, nl_ref, n_ref, m_ref):
    a = a_ref[0]
    x = x_ref[0]
    n_ref[0] = jnp.dot(a.astype(jnp.bfloat16), x.astype(jnp.bfloat16),
                       preferred_element_type=jnp.float32)
    pos = lax.broadcasted_iota(jnp.int32, (1, 1, NODE), 2)
    m_ref[...] = (pos < nl_ref[0]).astype(jnp.float32)


def _context(A3, seq_output, node_lengths):
    return pl.pallas_call(
        _ctx_body,
        grid=(B,),
        in_specs=[
            pl.BlockSpec((1, NODE, S), lambda b: (b, 0, 0)),
            pl.BlockSpec((1, S, D), lambda b: (b, 0, 0)),
            pl.BlockSpec(memory_space=pltpu.SMEM),
        ],
        out_specs=[
            pl.BlockSpec((1, NODE, D), lambda b: (b, 0, 0)),
            pl.BlockSpec((1, 1, NODE), lambda b: (b, 0, 0)),
        ],
        out_shape=[
            jax.ShapeDtypeStruct((B, NODE, D), jnp.float32),
            jax.ShapeDtypeStruct((B, 1, NODE), jnp.float32),
        ],
    )(A3, seq_output, node_lengths)


def kernel(seq_output, hidden, index, lengths, node_lengths, feat_seqs,
           node_type, W_pre, b_pre, W_q, W_v, max_length):
    hidden2 = jnp.transpose(hidden, (1, 0, 2)).reshape(B, D)
    e3 = _energy(seq_output, W_pre, b_pre.reshape(1, DK),
                 hidden2.reshape(B, 1, D), W_q, W_v)
    e_flat = e3.reshape(B * S)
    idx_t = index.reshape(NW, BAGS_PER_W * L).astype(jnp.int32)
    A = _sc_attention(e_flat, idx_t)
    A3 = A.reshape(B, NODE, S)
    nodes, mask3 = _context(A3, seq_output, node_lengths)
    return nodes, mask3.reshape(B, NODE), hidden2


# speedup vs baseline: 1.0188x; 1.0188x over previous
import jax
import jax.numpy as jnp
from jax import lax
from jax.experimental import pallas as pl
from jax.experimental.pallas import tpu as pltpu
from jax.experimental.pallas import tpu_sc as plsc

B = 8
S = 2048
D = 2048
DK = 256
NODE = 64
L = 64
N = B * NODE

NC = 2
NS = 16
NW = NC * NS
LANES = 16

BAGS_PER_W = N // NW
SBLK = 1024

_MESH = plsc.VectorSubcoreMesh(core_axis_name="c", subcore_axis_name="s",
                               num_cores=NC, num_subcores=NS)


def _energy_body(x_ref, w_ref, b_ref, h_ref, wq_ref, wv_ref, e_ref):
    x = x_ref[0].astype(jnp.bfloat16)
    w = w_ref[...].astype(jnp.bfloat16)
    p = jnp.dot(x, w, preferred_element_type=jnp.float32)
    t = jnp.dot(h_ref[0].astype(jnp.bfloat16),
                wq_ref[...].astype(jnp.bfloat16),
                preferred_element_type=jnp.float32)
    tmp = jnp.tanh(p + b_ref[...] + t)
    e_ref[0] = jnp.sum(tmp * wv_ref[...], axis=1, keepdims=True)


def _energy(seq_output, W_pre, b_pre2, hidden3, W_q, wv2):
    return pl.pallas_call(
        _energy_body,
        grid=(B, S // SBLK),
        in_specs=[
            pl.BlockSpec((1, SBLK, D), lambda b, s: (b, s, 0)),
            pl.BlockSpec((D, DK), lambda b, s: (0, 0)),
            pl.BlockSpec((1, DK), lambda b, s: (0, 0)),
            pl.BlockSpec((1, 1, D), lambda b, s: (b, 0, 0)),
            pl.BlockSpec((D, DK), lambda b, s: (0, 0)),
            pl.BlockSpec((1, DK), lambda b, s: (0, 0)),
        ],
        out_specs=pl.BlockSpec((1, SBLK, 1), lambda b, s: (b, s, 0)),
        out_shape=jax.ShapeDtypeStruct((B, S, 1), jnp.float32),
        compiler_params=pltpu.CompilerParams(
            vmem_limit_bytes=100 * 1024 * 1024),
    )(seq_output, W_pre, b_pre2, hidden3, W_q, wv2)


def _sc_attn_body(e_flat, idx_t, a_out, e_row, ix_v, sc_v, acc):
    wid = lax.axis_index("s") * NC + lax.axis_index("c")
    sample = wid // (NW // B)
    pltpu.sync_copy(e_flat.at[pl.ds(sample * S, S)], e_row)
    pltpu.sync_copy(idx_t.at[wid], ix_v)
    zeros = jnp.zeros((LANES,), jnp.float32)

    def zero_row(j, _):
        for i in range(LANES):
            acc[i, pl.ds(j * LANES, LANES)] = zeros
        return 0

    lax.fori_loop(0, S // LANES, zero_row, 0)

    def bag_softmax(i, _):
        base = i * L
        ev = [plsc.load_gather(e_row, [ix_v[pl.ds(base + k * LANES, LANES)]])
              for k in range(L // LANES)]
        m = ev[0]
        for v in ev[1:]:
            m = jnp.maximum(m, v)
        mm = jnp.max(m)
        pv = [jnp.exp(v - mm) for v in ev]
        ssum = pv[0]
        for v in pv[1:]:
            ssum = ssum + v
        rv = (zeros + 1.0) / (zeros + jnp.sum(ssum))
        for k in range(L // LANES):
            sc_v[pl.ds(base + k * LANES, LANES)] = pv[k] * rv
        return 0

    lax.fori_loop(0, BAGS_PER_W, bag_softmax, 0)

    rows16 = lax.iota(jnp.int32, LANES)
    pos0 = rows16 * L

    def scatter_l(l, _):
        pos = pos0 + l
        iv = plsc.load_gather(ix_v, [pos])
        sv = plsc.load_gather(sc_v, [pos])
        plsc.addupdate_scatter(acc, [rows16, iv], sv)
        return 0

    lax.fori_loop(0, L, scatter_l, 0)
    pltpu.sync_copy(acc, a_out.at[pl.ds(wid * BAGS_PER_W, BAGS_PER_W)])


def _sc_attention(e_flat, idx_t):
    k = pl.kernel(
        _sc_attn_body,
        out_type=jax.ShapeDtypeStruct((N, S), jnp.float32),
        mesh=_MESH,
        scratch_types=[
            pltpu.VMEM((S,), jnp.float32),
            pltpu.VMEM((BAGS_PER_W * L,), jnp.int32),
            pltpu.VMEM((BAGS_PER_W * L,), jnp.float32),
            pltpu.VMEM((LANES, S), jnp.float32),
        ],
        compiler_params=pltpu.CompilerParams(use_tc_tiling_on_sc=False,
                                             needs_layout_passes=False,
                                             disable_bounds_checks=True),
    )
    return k(e_flat, idx_t)


def _ctx_body(a_ref, x_ref, nl_ref, n_ref, m_ref):
    a = a_ref[0]
    x = x_ref[0]
    n_ref[0] = jnp.dot(a.astype(jnp.bfloat16), x.astype(jnp.bfloat16),
                       preferred_element_type=jnp.float32)
    pos = lax.broadcasted_iota(jnp.int32, (1, 1, NODE), 2)
    m_ref[...] = (pos < nl_ref[0]).astype(jnp.float32)


def _context(A3, seq_output, node_lengths):
    return pl.pallas_call(
        _ctx_body,
        grid=(B,),
        in_specs=[
            pl.BlockSpec((1, NODE, S), lambda b: (b, 0, 0)),
            pl.BlockSpec((1, S, D), lambda b: (b, 0, 0)),
            pl.BlockSpec(memory_space=pltpu.SMEM),
        ],
        out_specs=[
            pl.BlockSpec((1, NODE, D), lambda b: (b, 0, 0)),
            pl.BlockSpec((1, 1, NODE), lambda b: (b, 0, 0)),
        ],
        out_shape=[
            jax.ShapeDtypeStruct((B, NODE, D), jnp.float32),
            jax.ShapeDtypeStruct((B, 1, NODE), jnp.float32),
        ],
    )(A3, seq_output, node_lengths)


def kernel(seq_output, hidden, index, lengths, node_lengths, feat_seqs,
           node_type, W_pre, b_pre, W_q, W_v, max_length):
    hidden2 = jnp.transpose(hidden, (1, 0, 2)).reshape(B, D)
    e3 = _energy(seq_output, W_pre, b_pre.reshape(1, DK),
                 hidden2.reshape(B, 1, D), W_q, W_v.reshape(1, DK))
    e_flat = e3.reshape(B * S)
    idx_t = index.reshape(NW, BAGS_PER_W * L).astype(jnp.int32)
    A = _sc_attention(e_flat, idx_t)
    A3 = A.reshape(B, NODE, S)
    nodes, mask3 = _context(A3, seq_output, node_lengths)
    return nodes, mask3.reshape(B, NODE), hidden2
